# Initial kernel scaffold; baseline (speedup 1.0000x reference)
import functools

import jax
import jax.numpy as jnp
from jax import lax
from jax.experimental import pallas as pl
from jax.experimental.pallas import tpu as pltpu
from jax.experimental.pallas import tpu_sc as plsc

LANES = 16
NW = 32


def _dense_relu_kernel(x_ref, w_ref, b_ref, o_ref):
    acc = jnp.dot(x_ref[...], w_ref[...], preferred_element_type=jnp.float32)
    o_ref[...] = jnp.maximum(acc + b_ref[...], 0.0)


def _dense_relu(x, W, b, block_rows):
    n, d = x.shape
    h = W.shape[1]
    assert n % block_rows == 0
    return pl.pallas_call(
        _dense_relu_kernel,
        grid=(n // block_rows,),
        in_specs=[
            pl.BlockSpec((block_rows, d), lambda i: (i, 0)),
            pl.BlockSpec((d, h), lambda i: (0, 0)),
            pl.BlockSpec((1, h), lambda i: (0, 0)),
        ],
        out_specs=pl.BlockSpec((block_rows, h), lambda i: (i, 0)),
        out_shape=jax.ShapeDtypeStruct((n, h), jnp.float32),
    )(x, W, b.reshape(1, h))


def _make_accumulate(n_pad, K, H, per_w, C):
    assert per_w * NW == n_pad and per_w % C == 0
    n_chunks = per_w // C
    G = (C * K) // 128
    assert G * 128 == C * K
    HV = H // LANES
    mesh = plsc.VectorSubcoreMesh(core_axis_name="c", subcore_axis_name="s")

    @functools.partial(
        pl.kernel,
        out_type=jax.ShapeDtypeStruct((n_pad, 2 * H), jnp.float32),
        mesh=mesh,
        scratch_types=[
            pltpu.VMEM((G, 128), jnp.int32),
            pltpu.VMEM((C * K,), jnp.float32),
            pltpu.VMEM((C, H), jnp.float32),
            pltpu.VMEM((C * K, H), jnp.float32),
            pltpu.VMEM((C, 2 * H), jnp.float32),
            pltpu.VMEM((LANES,), jnp.float32),
            pltpu.SemaphoreType.DMA,
        ],
    )
    def acc(h_hbm, idx_hbm, dsq_hbm, out_hbm,
            idx_v, dsq_v, own_v, rows_v, out_v, w_s, sem):
        wid = lax.axis_index("s") * 2 + lax.axis_index("c")
        base0 = wid * per_w

        def chunk_body(c, carry):
            base = base0 + c * C
            pltpu.sync_copy(idx_hbm.at[pl.ds(base * K // 128, G)], idx_v)
            pltpu.sync_copy(dsq_hbm.at[pl.ds(base * K, C * K)], dsq_v)
            pltpu.sync_copy(h_hbm.at[pl.ds(base, C)], own_v)
            for g in range(G):
                pltpu.async_copy(h_hbm.at[idx_v.at[g]],
                                 rows_v.at[pl.ds(g * 128, 128)], sem)
            for g in range(G):
                pltpu.make_async_copy(h_hbm.at[idx_v.at[g]],
                                      rows_v.at[pl.ds(g * 128, 128)], sem).wait()

            def node_body(i, carry2):
                wvec = jnp.exp(dsq_v[pl.ds(i * K, K)] * (-10.0))
                w_s[...] = wvec
                rb = i * K
                s = [jnp.zeros((LANES,), jnp.float32) for _ in range(HV)]
                m = [jnp.full((LANES,), -jnp.inf, jnp.float32)
                     for _ in range(HV)]
                for k in range(K):
                    wk = w_s[k]
                    for j in range(HV):
                        wg = rows_v[rb + k, pl.ds(j * LANES, LANES)] * wk
                        s[j] = s[j] + wg
                        m[j] = jnp.maximum(m[j], wg)
                for j in range(HV):
                    o = own_v[i, pl.ds(j * LANES, LANES)]
                    out_v[i, pl.ds(j * LANES, LANES)] = s[j] * (1.0 / K) - o
                    out_v[i, pl.ds(H + j * LANES, LANES)] = m[j] - o
                return carry2

            lax.fori_loop(0, C, node_body, 0)
            pltpu.sync_copy(out_v, out_hbm.at[pl.ds(base, C)])
            return carry

        lax.fori_loop(0, n_chunks, chunk_body, 0)

    return acc


def kernel(x, neighbor_indices, distancesq, W0, b0, W1, b1):
    n, d = x.shape
    K = neighbor_indices.shape[1]
    H = W0.shape[1]

    C = 56
    per_w = -(-n // (NW * C)) * C
    n_pad = per_w * NW

    pad_n = n_pad - n
    x_pad = jnp.pad(x, ((0, pad_n), (0, 0)))
    idx2d = jnp.pad(neighbor_indices, ((0, pad_n), (0, 0))).reshape(-1, 128)
    dsq_flat = jnp.pad(distancesq, ((0, pad_n), (0, 0))).reshape(-1)

    acc = _make_accumulate(n_pad, K, H, per_w, C)

    h0 = _dense_relu(x_pad, W0, b0, block_rows=448)
    f1 = acc(h0, idx2d, dsq_flat)
    h1 = _dense_relu(f1, W1, b1, block_rows=448)
    f2 = acc(h1, idx2d, dsq_flat)
    return jnp.concatenate([f1[:n], f2[:n], x], axis=-1)


# speedup vs baseline: 3.2147x; 3.2147x over previous
import functools

import jax
import jax.numpy as jnp
from jax import lax
from jax.experimental import pallas as pl
from jax.experimental.pallas import tpu as pltpu
from jax.experimental.pallas import tpu_sc as plsc

LANES = 16
NW = 32


def _dense_relu_kernel(x_ref, w_ref, b_ref, o_ref):
    acc = jnp.dot(x_ref[...], w_ref[...], preferred_element_type=jnp.float32)
    o_ref[...] = jnp.maximum(acc + b_ref[...], 0.0)


def _dense_relu(x, W, b, block_rows):
    n, d = x.shape
    h = W.shape[1]
    assert n % block_rows == 0
    return pl.pallas_call(
        _dense_relu_kernel,
        grid=(n // block_rows,),
        in_specs=[
            pl.BlockSpec((block_rows, d), lambda i: (i, 0)),
            pl.BlockSpec((d, h), lambda i: (0, 0)),
            pl.BlockSpec((1, h), lambda i: (0, 0)),
        ],
        out_specs=pl.BlockSpec((block_rows, h), lambda i: (i, 0)),
        out_shape=jax.ShapeDtypeStruct((n, h), jnp.float32),
    )(x, W, b.reshape(1, h))


def _make_accumulate(n_pad, K, H, per_w, C):
    assert per_w * NW == n_pad and per_w % C == 0
    n_chunks = per_w // C
    G = (C * K) // 128
    assert G * 128 == C * K
    HV = H // LANES
    mesh = plsc.VectorSubcoreMesh(core_axis_name="c", subcore_axis_name="s")

    @functools.partial(
        pl.kernel,
        out_type=jax.ShapeDtypeStruct((n_pad, 2 * H), jnp.float32),
        mesh=mesh,
        compiler_params=pltpu.CompilerParams(use_tc_tiling_on_sc=False),
        scratch_types=[
            pltpu.VMEM((G, 128), jnp.int32),
            pltpu.VMEM((C * K,), jnp.float32),
            pltpu.VMEM((C, H), jnp.float32),
            pltpu.VMEM((C * K, H), jnp.float32),
            pltpu.VMEM((C, 2 * H), jnp.float32),
            pltpu.SemaphoreType.DMA,
        ],
    )
    def acc(h_hbm, idx_hbm, dsq_hbm, out_hbm,
            idx_v, dsq_v, own_v, rows_v, out_v, sem):
        wid = lax.axis_index("s") * 2 + lax.axis_index("c")
        base0 = wid * per_w

        def chunk_body(c, carry):
            base = pl.multiple_of(base0 + c * C, C)
            pltpu.sync_copy(idx_hbm.at[pl.ds(pl.multiple_of(base * K // 128, 8), G)], idx_v)
            pltpu.sync_copy(dsq_hbm.at[pl.ds(base * K, C * K)], dsq_v)
            pltpu.sync_copy(h_hbm.at[pl.ds(base, C)], own_v)
            for g in range(G):
                pltpu.async_copy(h_hbm.at[idx_v.at[g]],
                                 rows_v.at[pl.ds(g * 128, 128)], sem)
            for g in range(G):
                pltpu.make_async_copy(h_hbm.at[idx_v.at[g]],
                                      rows_v.at[pl.ds(g * 128, 128)], sem).wait()

            def node_body(i, carry2):
                wvec = jnp.exp(dsq_v[pl.ds(i * K, K)] * (-10.0))
                rb = i * K
                s = [jnp.zeros((LANES,), jnp.float32) for _ in range(HV)]
                m = [jnp.full((LANES,), -jnp.inf, jnp.float32)
                     for _ in range(HV)]
                for k in range(K):
                    wk = wvec[k]
                    for j in range(HV):
                        wg = rows_v[rb + k, pl.ds(j * LANES, LANES)] * wk
                        s[j] = s[j] + wg
                        m[j] = jnp.maximum(m[j], wg)
                for j in range(HV):
                    o = own_v[i, pl.ds(j * LANES, LANES)]
                    out_v[i, pl.ds(j * LANES, LANES)] = s[j] * (1.0 / K) - o
                    out_v[i, pl.ds(H + j * LANES, LANES)] = m[j] - o
                return carry2

            lax.fori_loop(0, C, node_body, 0)
            pltpu.sync_copy(out_v, out_hbm.at[pl.ds(base, C)])
            return carry

        lax.fori_loop(0, n_chunks, chunk_body, 0)

    return acc


def kernel(x, neighbor_indices, distancesq, W0, b0, W1, b1):
    n, d = x.shape
    K = neighbor_indices.shape[1]
    H = W0.shape[1]

    C = 64
    per_w = -(-n // (NW * C)) * C
    n_pad = per_w * NW

    pad_n = n_pad - n
    x_pad = jnp.pad(x, ((0, pad_n), (0, 0)))
    idx2d = jnp.pad(neighbor_indices, ((0, pad_n), (0, 0))).reshape(-1, 128)
    dsq_flat = jnp.pad(distancesq, ((0, pad_n), (0, 0))).reshape(-1)

    acc = _make_accumulate(n_pad, K, H, per_w, C)

    h0 = _dense_relu(x_pad, W0, b0, block_rows=512)
    f1 = acc(h0, idx2d, dsq_flat)
    h1 = _dense_relu(f1, W1, b1, block_rows=512)
    f2 = acc(h1, idx2d, dsq_flat)
    return jnp.concatenate([f1[:n], f2[:n], x], axis=-1)
